# baseline (speedup 1.0000x reference)
import jax
import jax.numpy as jnp
from jax.experimental import pallas as pl
from jax.experimental.pallas import tpu as pltpu


def _round_up(v, m):
    return ((v + m - 1) // m) * m


def _make_kernel(bounds):
    def _dense_kernel(x_ref, w_hbm_ref, b_ref, o_ref, w_vmem, sems):
        i = pl.program_id(0)
        nc = len(bounds) - 1

        def _chunk_copy(c):
            cols = pl.ds(bounds[c], bounds[c + 1] - bounds[c])
            return pltpu.make_async_copy(
                w_hbm_ref.at[:, cols], w_vmem.at[:, cols], sems.at[c])

        @pl.when(i == 0)
        def _first():
            for c in range(nc):
                _chunk_copy(c).start()
            for c in range(nc):
                _chunk_copy(c).wait()
                lo, hi = bounds[c], bounds[c + 1]
                acc = jnp.dot(x_ref[...], w_vmem[:, lo:hi],
                              preferred_element_type=jnp.float32)
                o_ref[:, lo:hi] = (
                    acc + b_ref[:, lo:hi].astype(jnp.float32)
                ).astype(o_ref.dtype)

        @pl.when(i > 0)
        def _rest():
            acc = jnp.dot(x_ref[...], w_vmem[...],
                          preferred_element_type=jnp.float32)
            o_ref[...] = (acc + b_ref[...].astype(jnp.float32)
                          ).astype(o_ref.dtype)

    return _dense_kernel


def kernel(x, w_kn, b):
    in_dim, out_dim = w_kn.shape
    orig_shape = x.shape
    out_dtype = x.dtype

    x2 = x.reshape(-1, in_dim)
    m = x2.shape[0]

    k_p = _round_up(in_dim, 128)
    n_p = _round_up(out_dim, 128)
    w_p = w_kn
    if (k_p, n_p) != (in_dim, out_dim):
        w_p = jnp.pad(w_kn, ((0, k_p - in_dim), (0, n_p - out_dim)))
    b_p = b
    if b.shape != (1, n_p):
        b_p = jnp.pad(b.reshape(1, -1), ((0, 0), (0, n_p - b.size)))

    tm = min(512, _round_up(m, 8))
    m_p = _round_up(m, tm)
    x_p = x2
    if (m_p, k_p) != (m, in_dim):
        x_p = jnp.pad(x2, ((0, m_p - m), (0, k_p - in_dim)))

    if n_p % 2048 == 0 and n_p >= 2048:
        u = n_p // 2048
        bounds = [0, 256 * u, 768 * u, 1408 * u, 2048 * u]
    else:
        bounds = [0, n_p]
    nc = len(bounds) - 1
    grid = (m_p // tm,)

    x_item = jnp.dtype(x_p.dtype).itemsize
    o_item = jnp.dtype(out_dtype).itemsize
    cost = pl.CostEstimate(
        flops=2 * m_p * k_p * n_p,
        transcendentals=0,
        bytes_accessed=(m_p * k_p * x_item + k_p * n_p * 4
                        + n_p * 4 + m_p * n_p * o_item),
    )

    out_p = pl.pallas_call(
        _make_kernel(bounds),
        out_shape=jax.ShapeDtypeStruct((m_p, n_p), out_dtype),
        grid=grid,
        in_specs=[
            pl.BlockSpec((tm, k_p), lambda i: (i, 0)),
            pl.BlockSpec(memory_space=pl.ANY),
            pl.BlockSpec((1, n_p), lambda i: (0, 0)),
        ],
        out_specs=pl.BlockSpec((tm, n_p), lambda i: (i, 0)),
        scratch_shapes=[
            pltpu.VMEM((k_p, n_p), jnp.float32),
            pltpu.SemaphoreType.DMA((nc,)),
        ],
        compiler_params=pltpu.CompilerParams(
            dimension_semantics=("arbitrary",),
            vmem_limit_bytes=60 * 1024 * 1024,
        ),
        cost_estimate=cost,
    )(x_p, w_p, b_p)

    out = out_p[:m, :out_dim]
    return out.reshape(orig_shape[:-1] + (out_dim,))


# speedup vs baseline: 1.0351x; 1.0351x over previous
import jax
import jax.numpy as jnp
from jax.experimental import pallas as pl
from jax.experimental.pallas import tpu as pltpu


def _round_up(v, m):
    return ((v + m - 1) // m) * m


def _make_kernel(nc, hn):
    def _dense_kernel(x_ref, w_hbm_ref, b_ref, o_ref, w_vmem, sems):
        i = pl.program_id(0)

        def _chunk_copy(c):
            cols = pl.ds(c * hn, hn)
            return pltpu.make_async_copy(
                w_hbm_ref.at[:, cols], w_vmem.at[:, cols], sems.at[c])

        @pl.when(i == 0)
        def _first():
            for c in range(nc):
                _chunk_copy(c).start()
            for c in range(nc):
                _chunk_copy(c).wait()
                lo, hi = c * hn, (c + 1) * hn
                acc = jnp.dot(x_ref[...], w_vmem[:, lo:hi],
                              preferred_element_type=jnp.float32)
                o_ref[:, lo:hi] = (
                    acc + b_ref[:, lo:hi].astype(jnp.float32)
                ).astype(o_ref.dtype)

        @pl.when(i > 0)
        def _rest():
            acc = jnp.dot(x_ref[...], w_vmem[...],
                          preferred_element_type=jnp.float32)
            o_ref[...] = (acc + b_ref[...].astype(jnp.float32)
                          ).astype(o_ref.dtype)

    return _dense_kernel


def kernel(x, w_kn, b):
    in_dim, out_dim = w_kn.shape
    orig_shape = x.shape
    out_dtype = x.dtype

    x2 = x.reshape(-1, in_dim)
    m = x2.shape[0]

    k_p = _round_up(in_dim, 128)
    n_p = _round_up(out_dim, 128)
    w_p = w_kn
    if (k_p, n_p) != (in_dim, out_dim):
        w_p = jnp.pad(w_kn, ((0, k_p - in_dim), (0, n_p - out_dim)))
    b_p = b
    if b.shape != (1, n_p):
        b_p = jnp.pad(b.reshape(1, -1), ((0, 0), (0, n_p - b.size)))

    tm = min(512, _round_up(m, 8))
    m_p = _round_up(m, tm)
    x_p = x2
    if (m_p, k_p) != (m, in_dim):
        x_p = jnp.pad(x2, ((0, m_p - m), (0, k_p - in_dim)))

    nc = 4 if (n_p % 1024 == 0 and n_p >= 1024) else 1
    hn = n_p // nc
    grid = (m_p // tm,)

    x_item = jnp.dtype(x_p.dtype).itemsize
    o_item = jnp.dtype(out_dtype).itemsize
    cost = pl.CostEstimate(
        flops=2 * m_p * k_p * n_p,
        transcendentals=0,
        bytes_accessed=(m_p * k_p * x_item + k_p * n_p * 4
                        + n_p * 4 + m_p * n_p * o_item),
    )

    out_p = pl.pallas_call(
        _make_kernel(nc, hn),
        out_shape=jax.ShapeDtypeStruct((m_p, n_p), out_dtype),
        grid=grid,
        in_specs=[
            pl.BlockSpec((tm, k_p), lambda i: (i, 0)),
            pl.BlockSpec(memory_space=pl.ANY),
            pl.BlockSpec((1, n_p), lambda i: (0, 0)),
        ],
        out_specs=pl.BlockSpec((tm, n_p), lambda i: (i, 0)),
        scratch_shapes=[
            pltpu.VMEM((k_p, n_p), jnp.float32),
            pltpu.SemaphoreType.DMA((nc,)),
        ],
        compiler_params=pltpu.CompilerParams(
            dimension_semantics=("arbitrary",),
            vmem_limit_bytes=60 * 1024 * 1024,
        ),
        cost_estimate=cost,
    )(x_p, w_p, b_p)

    out = out_p[:m, :out_dim]
    return out.reshape(orig_shape[:-1] + (out_dim,))
